# baseline (speedup 1.0000x reference)
import functools

import jax
import jax.numpy as jnp
from jax import lax
from jax.experimental import pallas as pl
from jax.experimental.pallas import tpu as pltpu
from jax.experimental.pallas import tpu_sc as plsc


def _build(B, L, V, D, NC, NS):
  NW = NC * NS
  TOT = B * L
  ROWS_W = TOT // NW
  CHUNK = 128
  NCHUNK = ROWS_W // CHUNK

  mesh = plsc.VectorSubcoreMesh(core_axis_name="c", subcore_axis_name="s")

  @functools.partial(
      pl.kernel,
      mesh=mesh,
      compiler_params=pltpu.CompilerParams(use_tc_tiling_on_sc=False),
      out_type=jax.ShapeDtypeStruct((TOT, D), jnp.float32),
      scratch_types=[
          pltpu.VMEM((NCHUNK, CHUNK), jnp.int32),
          pltpu.VMEM((2 * L, D), jnp.float32),
          pltpu.VMEM((CHUNK, D), jnp.float32),
          pltpu.SemaphoreType.DMA,
      ],
  )
  def emb(idx_hbm, table_hbm, pos_hbm, out_hbm, idx_v, pos_v, buf, sem):
    wid = lax.axis_index("s") * NC + lax.axis_index("c")
    base = wid * ROWS_W
    pltpu.sync_copy(idx_hbm.at[wid], idx_v)
    pltpu.sync_copy(pos_hbm, pos_v.at[pl.ds(0, L)])
    pltpu.sync_copy(pos_hbm, pos_v.at[pl.ds(L, L)])

    def chunk_body(g, carry):
      pltpu.async_copy(table_hbm.at[idx_v.at[g]], buf, sem).wait()
      roff = lax.rem(g * CHUNK, L)
      def add_body(r, c2):
        for j in range(D // 16):
          vec = pos_v[roff + r, pl.ds(j * 16, 16)]
          plsc.addupdate(buf.at[r, pl.ds(j * 16, 16)], vec)
        return c2
      lax.fori_loop(0, CHUNK, add_body, 0)
      pltpu.sync_copy(buf, out_hbm.at[pl.ds(base + g * CHUNK, CHUNK)])
      return carry

    lax.fori_loop(0, NCHUNK, chunk_body, 0)

  return emb


def kernel(x, token_table, pos_table):
  B, L = x.shape
  V, D = token_table.shape
  info = plsc.get_sparse_core_info()
  NC, NS = info.num_cores, info.num_subcores
  NW = NC * NS
  ROWS_W = (B * L) // NW
  CHUNK = 128
  idx = x.astype(jnp.int32).reshape(NW, ROWS_W // CHUNK, CHUNK)
  out = _build(B, L, V, D, NC, NS)(idx, token_table, pos_table)
  return out.reshape(B, L, D)


# speedup vs baseline: 1.1129x; 1.1129x over previous
import functools

import jax
import jax.numpy as jnp
from jax import lax
from jax.experimental import pallas as pl
from jax.experimental.pallas import tpu as pltpu
from jax.experimental.pallas import tpu_sc as plsc


def _build(B, L, V, D, NC, NS):
  NW = NC * NS
  TOT = B * L
  ROWS_W = TOT // NW
  CHUNK = 128
  NCHUNK = ROWS_W // CHUNK

  mesh = plsc.VectorSubcoreMesh(core_axis_name="c", subcore_axis_name="s")

  @functools.partial(
      pl.kernel,
      mesh=mesh,
      compiler_params=pltpu.CompilerParams(use_tc_tiling_on_sc=False),
      out_type=jax.ShapeDtypeStruct((TOT, D), jnp.float32),
      scratch_types=[
          pltpu.VMEM((NCHUNK, CHUNK), jnp.int32),
          pltpu.VMEM((2 * L, D), jnp.float32),
          pltpu.VMEM((CHUNK, D), jnp.float32),
          pltpu.VMEM((CHUNK, D), jnp.float32),
          pltpu.VMEM((CHUNK, D), jnp.float32),
          pltpu.VMEM((CHUNK, D), jnp.float32),
          pltpu.SemaphoreType.DMA,
          pltpu.SemaphoreType.DMA,
          pltpu.SemaphoreType.DMA,
          pltpu.SemaphoreType.DMA,
      ],
  )
  def emb(idx_hbm, table_hbm, pos_hbm, out_hbm,
          idx_v, pos_v, g0, g1, o0, o1, gs0, gs1, os0, os1):
    bufs, obufs, gsems, osems = [g0, g1], [o0, o1], [gs0, gs1], [os0, os1]
    wid = lax.axis_index("s") * NC + lax.axis_index("c")
    base = wid * ROWS_W
    pltpu.sync_copy(idx_hbm.at[wid], idx_v)
    pltpu.sync_copy(pos_hbm, pos_v.at[pl.ds(0, L)])
    pltpu.sync_copy(pos_hbm, pos_v.at[pl.ds(L, L)])

    pltpu.async_copy(table_hbm.at[idx_v.at[0]], g0, gs0)
    pltpu.async_copy(table_hbm.at[idx_v.at[1]], g1, gs1)

    def step(k, carry):
      for b in range(2):
        c = 2 * k + b
        pltpu.make_async_copy(
            table_hbm.at[idx_v.at[c]], bufs[b], gsems[b]).wait()
        @pl.when(k > 0)
        def _():
          pltpu.make_async_copy(
              obufs[b], out_hbm.at[pl.ds(base, CHUNK)], osems[b]).wait()
        roff = lax.rem(c * CHUNK, L)
        def add_body(r2, c2):
          r = r2 * 2
          for rr in range(2):
            for j in range(D // 16):
              sl = pl.ds(j * 16, 16)
              obufs[b][r + rr, sl] = (
                  bufs[b][r + rr, sl] + pos_v[roff + r + rr, sl])
          return c2
        lax.fori_loop(0, CHUNK // 2, add_body, 0)
        pltpu.async_copy(
            obufs[b], out_hbm.at[pl.ds(base + c * CHUNK, CHUNK)], osems[b])
        @pl.when(k < NCHUNK // 2 - 1)
        def _():
          pltpu.async_copy(table_hbm.at[idx_v.at[c + 2]], bufs[b], gsems[b])
      return carry

    lax.fori_loop(0, NCHUNK // 2, step, 0)
    for b in range(2):
      pltpu.make_async_copy(
          obufs[b], out_hbm.at[pl.ds(base, CHUNK)], osems[b]).wait()

  return emb


def kernel(x, token_table, pos_table):
  B, L = x.shape
  V, D = token_table.shape
  info = plsc.get_sparse_core_info()
  NC, NS = info.num_cores, info.num_subcores
  NW = NC * NS
  ROWS_W = (B * L) // NW
  CHUNK = 128
  idx = x.astype(jnp.int32).reshape(NW, ROWS_W // CHUNK, CHUNK)
  out = _build(B, L, V, D, NC, NS)(idx, token_table, pos_table)
  return out.reshape(B, L, D)


# speedup vs baseline: 1.4484x; 1.3015x over previous
import functools

import jax
import jax.numpy as jnp
from jax import lax
from jax.experimental import pallas as pl
from jax.experimental.pallas import tpu as pltpu
from jax.experimental.pallas import tpu_sc as plsc


def _build(B, L, V, D, NC, NS):
  NW = NC * NS
  BW = B // NW
  SPANS = ((0, 96), (96, L - 96))

  mesh = plsc.VectorSubcoreMesh(core_axis_name="c", subcore_axis_name="s")

  @functools.partial(
      pl.kernel,
      mesh=mesh,
      compiler_params=pltpu.CompilerParams(use_tc_tiling_on_sc=False),
      out_type=jax.ShapeDtypeStruct((B, L, D), jnp.float32),
      scratch_types=[
          pltpu.VMEM((BW, L), jnp.int32),
          pltpu.VMEM((L, D), jnp.float32),
          pltpu.VMEM((L, D), jnp.float32),
          pltpu.VMEM((L, D), jnp.float32),
          pltpu.VMEM((L, D), jnp.float32),
          pltpu.VMEM((L, D), jnp.float32),
          pltpu.SemaphoreType.DMA,
          pltpu.SemaphoreType.DMA,
          pltpu.SemaphoreType.DMA,
          pltpu.SemaphoreType.DMA,
      ],
  )
  def emb(x_hbm, table_hbm, pos_hbm, out_hbm,
          idx_v, pos_v, g0, g1, o0, o1, gs0, gs1, os0, os1):
    bufs, obufs, gsems, osems = [g0, g1], [o0, o1], [gs0, gs1], [os0, os1]
    wid = lax.axis_index("s") * NC + lax.axis_index("c")
    wbase = wid * BW
    pltpu.sync_copy(x_hbm.at[pl.ds(wbase, BW)], idx_v)
    pltpu.sync_copy(pos_hbm, pos_v)

    def gathers(bi, p):
      return [
          pltpu.make_async_copy(
              table_hbm.at[idx_v.at[bi, pl.ds(off, n)]],
              bufs[p].at[pl.ds(off, n)], gsems[p])
          for off, n in SPANS
      ]

    def writeback(bi, p):
      return pltpu.make_async_copy(obufs[p], out_hbm.at[wbase + bi], osems[p])

    for p in range(2):
      for c in gathers(p, p):
        c.start()

    def step(k, carry):
      for p in range(2):
        bi = 2 * k + p
        for c in gathers(bi, p):
          c.wait()
        @pl.when(k > 0)
        def _():
          writeback(bi - 2, p).wait()
        def add_body(r2, c2):
          r = r2 * 2
          for rr in range(2):
            for j in range(D // 16):
              sl = pl.ds(j * 16, 16)
              obufs[p][r + rr, sl] = bufs[p][r + rr, sl] + pos_v[r + rr, sl]
          return c2
        lax.fori_loop(0, L // 2, add_body, 0)
        writeback(bi, p).start()
        @pl.when(k < BW // 2 - 1)
        def _():
          for c in gathers(bi + 2, p):
            c.start()
      return carry

    lax.fori_loop(0, BW // 2, step, 0)
    for p in range(2):
      writeback(BW - 2 + p, p).wait()

  return emb


def kernel(x, token_table, pos_table):
  B, L = x.shape
  V, D = token_table.shape
  info = plsc.get_sparse_core_info()
  NC, NS = info.num_cores, info.num_subcores
  return _build(B, L, V, D, NC, NS)(
      x.astype(jnp.int32), token_table, pos_table)
